# baseline (speedup 1.0000x reference)
import jax
import jax.numpy as jnp
from jax.experimental import pallas as pl
from jax.experimental.pallas import tpu as pltpu

R = 4
N = 4096
DIN = 256
DOUT = 256
BM = 128
GRP = 2
NSLOT = GRP * R
MT = N // BM


def _rgcn_body(x_ref, w_ref, b_ref, adj_hbm, out_ref, sup_ref, abuf, sem):
    def start_dma(r, m, slot):
        pltpu.make_async_copy(
            adj_hbm.at[r, pl.ds(m * BM, BM), :],
            abuf.at[slot],
            sem.at[slot],
        ).start()

    for g in range(GRP):
        for r in range(R):
            start_dma(r, g, g * R + r)

    for r in range(R):
        s = jax.lax.dot_general(
            x_ref[...], w_ref[r], (((1,), (1,)), ((), ())),
            preferred_element_type=jnp.float32)
        sup_ref[r] = s + b_ref[r]

    def wait(slot):
        pltpu.make_async_copy(
            adj_hbm.at[0, pl.ds(0, BM), :], abuf.at[slot], sem.at[slot]
        ).wait()

    def body(m, carry):
        base = jax.lax.rem(m, GRP) * R
        wait(base)
        acc = jnp.dot(abuf[base], sup_ref[0],
                      preferred_element_type=jnp.float32)
        for r in range(1, R):
            wait(base + r)
            acc = acc + jnp.dot(abuf[base + r], sup_ref[r],
                                preferred_element_type=jnp.float32)
        out_ref[pl.ds(m * BM, BM), :] = jnp.tanh(acc)

        @pl.when(m + GRP < MT)
        def _():
            for r in range(R):
                start_dma(r, m + GRP, base + r)

        return carry

    jax.lax.fori_loop(0, MT, body, 0)


@jax.jit
def kernel(input, adjs, W, b):
    b3 = b.reshape(R, 1, DOUT)
    return pl.pallas_call(
        _rgcn_body,
        in_specs=[
            pl.BlockSpec((N, DIN), lambda: (0, 0)),
            pl.BlockSpec((R, DOUT, DIN), lambda: (0, 0, 0)),
            pl.BlockSpec((R, 1, DOUT), lambda: (0, 0, 0)),
            pl.BlockSpec(memory_space=pl.ANY),
        ],
        out_specs=pl.BlockSpec((N, DOUT), lambda: (0, 0)),
        out_shape=jax.ShapeDtypeStruct((N, DOUT), jnp.float32),
        scratch_shapes=[
            pltpu.VMEM((R, N, DOUT), jnp.float32),
            pltpu.VMEM((NSLOT, BM, N), jnp.float32),
            pltpu.SemaphoreType.DMA((NSLOT,)),
        ],
        compiler_params=pltpu.CompilerParams(
            vmem_limit_bytes=100 * 1024 * 1024,
        ),
    )(input, W, b3, adjs)
